# baseline (speedup 1.0000x reference)
import functools

import jax
import jax.numpy as jnp
from jax import lax
from jax.experimental import pallas as pl
from jax.experimental.pallas import tpu as pltpu
from jax.experimental.pallas import tpu_sc as plsc

N = 10000
NPAD = 10240
E = 320000
NC, NS = 2, 16
NW = NC * NS
CHUNK = 128
CPW = 80
EPAD = NW * CPW * CHUNK
RPT = NPAD // NS
H1, C1 = 8, 8
C2 = 10

_f32 = jnp.float32
_i32 = jnp.int32


@functools.lru_cache(maxsize=None)
def _sc_mesh():
  return plsc.VectorSubcoreMesh(
      core_axis_name="c", subcore_axis_name="s", num_cores=NC, num_subcores=NS)


def _sc_compiler_params():
  return pltpu.CompilerParams(
      use_tc_tiling_on_sc=False, needs_layout_passes=False)


def _iota16():
  return lax.iota(_i32, 16)


def _splat16(v):
  return jnp.broadcast_to(v, (16,)).astype(_i32)


def _zero_rows(buf, nrows, width_groups):
  z = jnp.zeros((16,), _f32)

  def body(i, _):
    if width_groups is None:
      buf[i] = z
    else:
      for g in range(width_groups):
        buf[i, g] = z
    return 0

  lax.fori_loop(0, nrows, body, 0)


def _fake_wait(src_hbm_like, dst_buf, sem):
  pltpu.make_async_copy(src_hbm_like, dst_buf, sem).wait()


def _run_pipeline(start_gathers, wait_gathers, compute, start_scatter,
                  wait_scatter):
  for b in range(2):
    start_gathers(b, b)

  def loop_body(j, _):
    for b in range(2):
      q = 2 * j + b
      wait_gathers(b)

      @pl.when(j > 0)
      def _():
        wait_scatter(b)

      compute(b)
      start_scatter(q, b)
      start_gathers(q + 2, b)
    return 0

  lax.fori_loop(0, CPW // 2 - 1, loop_body, 0)
  for b in range(2):
    q = CPW - 2 + b
    wait_gathers(b)
    wait_scatter(b)
    compute(b)
    start_scatter(q, b)
  for b in range(2):
    wait_scatter(b)


@functools.lru_cache(maxsize=None)
def _make_sc_edge1():
  @functools.partial(
      pl.kernel,
      out_type=(jax.ShapeDtypeStruct((NC, NPAD, 16), _f32),
                jax.ShapeDtypeStruct((NC, NPAD, 4, 16), _f32)),
      mesh=_sc_mesh(),
      compiler_params=_sc_compiler_params(),
      scratch_types=[
          pltpu.VMEM((CPW, CHUNK), _i32),
          pltpu.VMEM((CPW, CHUNK), _i32),
          pltpu.VMEM((CHUNK, 16), _f32),
          pltpu.VMEM((CHUNK, 16), _f32),
          pltpu.VMEM((CHUNK, 16), _f32),
          pltpu.VMEM((CHUNK, 16), _f32),
          pltpu.VMEM((CHUNK, 2, 16), _i32),
          pltpu.VMEM((CHUNK, 2, 16), _i32),
          pltpu.VMEM((CHUNK, 16), _f32),
          pltpu.VMEM((CHUNK, 16), _f32),
          pltpu.VMEM((CHUNK, 4, 16), _f32),
          pltpu.VMEM((CHUNK, 4, 16), _f32),
          pltpu.VMEM_SHARED((NPAD, 16), _f32),
          pltpu.VMEM_SHARED((NPAD, 4, 16), _f32),
          pltpu.SemaphoreType.DMA,
          pltpu.SemaphoreType.DMA,
          pltpu.SemaphoreType.DMA,
          pltpu.SemaphoreType.DMA,
          pltpu.SemaphoreType.DMA,
          pltpu.SemaphoreType.DMA,
      ],
  )
  def sc_edge1(src_hbm, dst_hbm, atab_hbm, btab_hbm, h1_hbm,
               dout_hbm, mout_hbm,
               sidx_all, didx_all, arows0, arows1, brows0, brows1,
               hrows0, hrows1, wrows0, wrows1, obuf0, obuf1,
               acc_d, acc_m, gsem0, gsem1, dsem0, dsem1, msem0, msem1):
    c = lax.axis_index("c")
    s = lax.axis_index("s")
    wid = s * NC + c
    slots = ((arows0, brows0, hrows0, wrows0, obuf0, gsem0, dsem0, msem0),
             (arows1, brows1, hrows1, wrows1, obuf1, gsem1, dsem1, msem1))

    _zero_rows(wrows0, CHUNK, None)
    _zero_rows(obuf0, CHUNK, 4)
    for b in range(RPT // CHUNK):
      pltpu.sync_copy(wrows0, acc_d.at[pl.ds(s * RPT + b * CHUNK, CHUNK)])
      pltpu.sync_copy(obuf0, acc_m.at[pl.ds(s * RPT + b * CHUNK, CHUNK)])

    pltpu.sync_copy(src_hbm.at[pl.ds(wid * CPW, CPW)], sidx_all)
    pltpu.sync_copy(dst_hbm.at[pl.ds(wid * CPW, CPW)], didx_all)
    plsc.subcore_barrier()

    io16 = _iota16()
    scale_base = io16 >> 3

    def start_gathers(q, b):
      ar, br, hr, gs = slots[b][0], slots[b][1], slots[b][2], slots[b][5]
      pltpu.async_copy(atab_hbm.at[sidx_all.at[q]], ar, gs)
      pltpu.async_copy(btab_hbm.at[didx_all.at[q]], br, gs)
      pltpu.async_copy(h1_hbm.at[sidx_all.at[q]], hr, gs)

    def wait_gathers(b):
      ar, br, hr, gs = slots[b][0], slots[b][1], slots[b][2], slots[b][5]
      _fake_wait(atab_hbm.at[pl.ds(0, CHUNK)], ar, gs)
      _fake_wait(btab_hbm.at[pl.ds(0, CHUNK)], br, gs)
      _fake_wait(h1_hbm.at[pl.ds(0, CHUNK)], hr, gs)

    def compute(b):
      ar, br, hr, wr, ob = (slots[b][0], slots[b][1], slots[b][2],
                            slots[b][3], slots[b][4])

      @plsc.parallel_loop(0, CHUNK, unroll=8)
      def _(k):
        kk = _splat16(k)
        t = ar[k] + br[k]
        wr[k] = jnp.exp(jnp.maximum(t, 0.2 * t))
        for g2 in range(2):
          pb = plsc.bitcast(hr[k, g2], jnp.bfloat16)
          lo, hi = plsc.unpack(pb, format=plsc.PackFormat.INTERLEAVED)
          s_lo = plsc.load_gather(wr, [kk, 4 * g2 + scale_base])
          s_hi = plsc.load_gather(wr, [kk, 4 * g2 + 2 + scale_base])
          ob[k, 2 * g2] = lo * s_lo
          ob[k, 2 * g2 + 1] = hi * s_hi

    def start_scatter(q, b):
      wr, ob, ds, ms = slots[b][3], slots[b][4], slots[b][6], slots[b][7]
      pltpu.async_copy(wr, acc_d.at[didx_all.at[q]], ds, add=True)
      pltpu.async_copy(ob, acc_m.at[didx_all.at[q]], ms, add=True)

    def wait_scatter(b):
      wr, ob, ds, ms = slots[b][3], slots[b][4], slots[b][6], slots[b][7]
      _fake_wait(atab_hbm.at[pl.ds(0, CHUNK)], wr, ds)
      _fake_wait(mout_hbm.at[0, pl.ds(0, CHUNK)], ob, ms)

    _run_pipeline(start_gathers, wait_gathers, compute, start_scatter,
                  wait_scatter)
    plsc.subcore_barrier()

    for b in range(RPT // CHUNK):
      r0 = s * RPT + b * CHUNK
      pltpu.sync_copy(acc_d.at[pl.ds(r0, CHUNK)], wrows0)
      pltpu.sync_copy(wrows0, dout_hbm.at[c, pl.ds(r0, CHUNK)])
      pltpu.sync_copy(acc_m.at[pl.ds(r0, CHUNK)], obuf0)
      pltpu.sync_copy(obuf0, mout_hbm.at[c, pl.ds(r0, CHUNK)])

  return sc_edge1


@functools.lru_cache(maxsize=None)
def _make_sc_edge2():
  @functools.partial(
      pl.kernel,
      out_type=(jax.ShapeDtypeStruct((NC, NPAD, 16), _f32),
                jax.ShapeDtypeStruct((NC, NPAD, 16), _f32)),
      mesh=_sc_mesh(),
      compiler_params=_sc_compiler_params(),
      scratch_types=[
          pltpu.VMEM((CPW, CHUNK), _i32),
          pltpu.VMEM((CPW, CHUNK), _i32),
          pltpu.VMEM((CHUNK, 16), _f32),
          pltpu.VMEM((CHUNK, 16), _f32),
          pltpu.VMEM((CHUNK, 16), _f32),
          pltpu.VMEM((CHUNK, 16), _f32),
          pltpu.VMEM((CHUNK, 16), _f32),
          pltpu.VMEM((CHUNK, 16), _f32),
          pltpu.VMEM((CHUNK, 16), _f32),
          pltpu.VMEM((CHUNK, 16), _f32),
          pltpu.VMEM_SHARED((NPAD, 16), _f32),
          pltpu.VMEM_SHARED((NPAD, 16), _f32),
          pltpu.SemaphoreType.DMA,
          pltpu.SemaphoreType.DMA,
          pltpu.SemaphoreType.DMA,
          pltpu.SemaphoreType.DMA,
          pltpu.SemaphoreType.DMA,
          pltpu.SemaphoreType.DMA,
      ],
  )
  def sc_edge2(src_hbm, dst_hbm, btab_hbm, h2tab_hbm,
               dout_hbm, mout_hbm,
               sidx_all, didx_all, brows0, brows1,
               hrows0, hrows1, wrows0, wrows1, obuf0, obuf1,
               acc_d, acc_m, gsem0, gsem1, dsem0, dsem1, msem0, msem1):
    c = lax.axis_index("c")
    s = lax.axis_index("s")
    wid = s * NC + c
    i15 = _splat16(15)
    slots = ((brows0, hrows0, wrows0, obuf0, gsem0, dsem0, msem0),
             (brows1, hrows1, wrows1, obuf1, gsem1, dsem1, msem1))

    _zero_rows(wrows0, CHUNK, None)
    for b in range(RPT // CHUNK):
      pltpu.sync_copy(wrows0, acc_d.at[pl.ds(s * RPT + b * CHUNK, CHUNK)])
      pltpu.sync_copy(wrows0, acc_m.at[pl.ds(s * RPT + b * CHUNK, CHUNK)])

    pltpu.sync_copy(src_hbm.at[pl.ds(wid * CPW, CPW)], sidx_all)
    pltpu.sync_copy(dst_hbm.at[pl.ds(wid * CPW, CPW)], didx_all)
    plsc.subcore_barrier()

    def start_gathers(q, b):
      br, hr, gs = slots[b][0], slots[b][1], slots[b][4]
      pltpu.async_copy(btab_hbm.at[didx_all.at[q]], br, gs)
      pltpu.async_copy(h2tab_hbm.at[sidx_all.at[q]], hr, gs)

    def wait_gathers(b):
      br, hr, gs = slots[b][0], slots[b][1], slots[b][4]
      _fake_wait(btab_hbm.at[pl.ds(0, CHUNK)], br, gs)
      _fake_wait(h2tab_hbm.at[pl.ds(0, CHUNK)], hr, gs)

    def compute(b):
      br, hr, wr, ob = (slots[b][0], slots[b][1], slots[b][2], slots[b][3])

      @plsc.parallel_loop(0, CHUNK, unroll=8)
      def _(k):
        kk = _splat16(k)
        asp = plsc.load_gather(hr, [kk, i15])
        t = asp + br[k]
        w = jnp.exp(jnp.maximum(t, 0.2 * t))
        wr[k] = w
        ob[k] = hr[k] * w

    def start_scatter(q, b):
      wr, ob, ds, ms = slots[b][2], slots[b][3], slots[b][5], slots[b][6]
      pltpu.async_copy(wr, acc_d.at[didx_all.at[q]], ds, add=True)
      pltpu.async_copy(ob, acc_m.at[didx_all.at[q]], ms, add=True)

    def wait_scatter(b):
      wr, ob, ds, ms = slots[b][2], slots[b][3], slots[b][5], slots[b][6]
      _fake_wait(btab_hbm.at[pl.ds(0, CHUNK)], wr, ds)
      _fake_wait(btab_hbm.at[pl.ds(0, CHUNK)], ob, ms)

    _run_pipeline(start_gathers, wait_gathers, compute, start_scatter,
                  wait_scatter)
    plsc.subcore_barrier()

    for b in range(RPT // CHUNK):
      r0 = s * RPT + b * CHUNK
      pltpu.sync_copy(acc_d.at[pl.ds(r0, CHUNK)], wrows0)
      pltpu.sync_copy(wrows0, dout_hbm.at[c, pl.ds(r0, CHUNK)])
      pltpu.sync_copy(acc_m.at[pl.ds(r0, CHUNK)], wrows0)
      pltpu.sync_copy(wrows0, mout_hbm.at[c, pl.ds(r0, CHUNK)])

  return sc_edge2


_BLK = 1024
_GRID = NPAD // _BLK


def _tc_spec(width):
  return pl.BlockSpec((_BLK, width), lambda i: (i, 0))


def _row_spec(width):
  return pl.BlockSpec((_BLK, width), lambda i: (0, 0))


def _full_spec(a):
  return pl.BlockSpec(a.shape, lambda i: tuple(0 for _ in a.shape))


def _k1_body(x_ref, w1_ref, as_ref, ad_ref,
             h1_ref, ph1_ref, s1tab_ref, adtab_ref, wself_ref):
  h = jnp.dot(x_ref[...], w1_ref[...], preferred_element_type=_f32)
  h1_ref[...] = h
  hb = h.astype(jnp.bfloat16)
  words = []
  for g2 in range(2):
    lo = lax.bitcast_convert_type(hb[:, 32 * g2:32 * g2 + 16],
                                  jnp.uint16).astype(jnp.uint32)
    hi = lax.bitcast_convert_type(hb[:, 32 * g2 + 16:32 * g2 + 32],
                                  jnp.uint16).astype(jnp.uint32)
    words.append(lo | (hi << 16))
  ph1_ref[...] = lax.bitcast_convert_type(
      jnp.concatenate(words, axis=1), jnp.int32)
  a_s = jnp.dot(h, as_ref[...], preferred_element_type=_f32)
  a_d = jnp.dot(h, ad_ref[...], preferred_element_type=_f32)
  s1tab_ref[...] = jnp.concatenate([a_s, a_s], axis=1)
  adtab_ref[...] = jnp.concatenate([a_d, a_d], axis=1)
  t = a_s + a_d
  wself_ref[...] = jnp.exp(jnp.maximum(t, 0.2 * t))


def _k5_body(dp_ref, mp_ref, h1_ref, wself_ref, b1_ref, w2_ref, a2s_ref,
             a2d_ref, r8_ref,
             x1_ref, h2tab_ref, b2tab_ref, wself2_ref):
  recip1 = 1.0 / (dp_ref[0][:, :8] + dp_ref[1][:, :8] + wself_ref[...] + 1e-16)
  r = jnp.dot(recip1, r8_ref[...], preferred_element_type=_f32)
  m = jnp.dot(wself_ref[...] * recip1, r8_ref[...],
              preferred_element_type=_f32)
  out1 = (mp_ref[0] + mp_ref[1]) * r + h1_ref[...] * m + b1_ref[...]
  x1 = jnp.where(out1 > 0, out1, jnp.exp(jnp.minimum(out1, 0.0)) - 1.0)
  x1_ref[...] = x1
  h2 = jnp.dot(x1, w2_ref[...], preferred_element_type=_f32)
  as2 = jnp.sum(h2 * a2s_ref[...], axis=1, keepdims=True)
  ad2 = jnp.sum(h2 * a2d_ref[...], axis=1, keepdims=True)
  lane = lax.broadcasted_iota(_i32, h2.shape, 1)
  h2tab_ref[...] = jnp.where(lane == 15, as2, h2)
  b2tab_ref[...] = jnp.broadcast_to(ad2, h2.shape)
  t = as2 + ad2
  wself2_ref[...] = jnp.broadcast_to(jnp.exp(jnp.maximum(t, 0.2 * t)), h2.shape)


def _k9_body(dp_ref, mp_ref, h2tab_ref, wself2_ref, b2_ref, out_ref):
  recip2 = 1.0 / (dp_ref[0] + dp_ref[1] + wself2_ref[...] + 1e-16)
  lane = lax.broadcasted_iota(_i32, recip2.shape, 1)
  h2 = jnp.where(lane == 15, 0.0, h2tab_ref[...])
  z = ((mp_ref[0] + mp_ref[1]) * recip2
       + h2 * (wself2_ref[...] * recip2) + b2_ref[...])
  valid = lane < C2
  zm = jnp.where(valid, z, -jnp.inf)
  mx = jnp.max(zm, axis=1, keepdims=True)
  ez = jnp.where(valid, jnp.exp(z - mx), 0.0)
  ssum = jnp.sum(ez, axis=1, keepdims=True)
  out_ref[...] = z - mx - jnp.log(ssum)


def kernel(x, edge_index, W1, a_src1, a_dst1, b1, W2, a_src2, a_dst2, b2):
  src = edge_index[0].astype(_i32)
  dst = edge_index[1].astype(_i32)
  pad_e = EPAD - E
  pad_idx = jnp.full((pad_e,), NPAD - 1, _i32)
  src_p = jnp.concatenate([src, pad_idx]).reshape(NW * CPW, CHUNK)
  dst_p = jnp.concatenate([dst, pad_idx]).reshape(NW * CPW, CHUNK)
  x_p = jnp.pad(x, ((0, NPAD - N), (0, 0)))

  eye8 = jnp.eye(H1, dtype=_f32)
  As1 = (a_src1[:, :, None] * eye8[:, None, :]).reshape(H1 * C1, H1)
  Ad1 = (a_dst1[:, :, None] * eye8[:, None, :]).reshape(H1 * C1, H1)
  R8 = (eye8[:, :, None] * jnp.ones((1, 1, C1), _f32)).reshape(H1, H1 * C1)
  b1_row = jnp.broadcast_to(b1[None, :], (_BLK, H1 * C1))
  W2p = jnp.pad(W2, ((0, 0), (0, 16 - C2)))
  a2s_row = jnp.broadcast_to(jnp.pad(a_src2[0], (0, 16 - C2))[None, :],
                             (_BLK, 16))
  a2d_row = jnp.broadcast_to(jnp.pad(a_dst2[0], (0, 16 - C2))[None, :],
                             (_BLK, 16))
  b2_row = jnp.broadcast_to(jnp.pad(b2, (0, 16 - C2))[None, :], (_BLK, 16))

  h1p, ph1, s1tab, adtab, wself1 = pl.pallas_call(
      _k1_body,
      grid=(_GRID,),
      in_specs=[_tc_spec(128), _full_spec(W1), _full_spec(As1), _full_spec(Ad1)],
      out_specs=[_tc_spec(64), _tc_spec(32), _tc_spec(16), _tc_spec(16),
                 _tc_spec(8)],
      out_shape=[
          jax.ShapeDtypeStruct((NPAD, 64), _f32),
          jax.ShapeDtypeStruct((NPAD, 32), _i32),
          jax.ShapeDtypeStruct((NPAD, 16), _f32),
          jax.ShapeDtypeStruct((NPAD, 16), _f32),
          jax.ShapeDtypeStruct((NPAD, 8), _f32),
      ],
  )(x_p, W1, As1, Ad1)

  denom1, msg1 = _make_sc_edge1()(src_p, dst_p, s1tab, adtab,
                                  ph1.reshape(NPAD, 2, 16))
  msg1 = msg1.reshape(NC, NPAD, 64)

  x1p, h2tab, b2tab, wself2 = pl.pallas_call(
      _k5_body,
      grid=(_GRID,),
      in_specs=[pl.BlockSpec((NC, _BLK, 16), lambda i: (0, i, 0)),
                pl.BlockSpec((NC, _BLK, 64), lambda i: (0, i, 0)),
                _tc_spec(64), _tc_spec(8), _row_spec(64),
                _full_spec(W2p), _row_spec(16), _row_spec(16), _full_spec(R8)],
      out_specs=[_tc_spec(64), _tc_spec(16), _tc_spec(16), _tc_spec(16)],
      out_shape=[
          jax.ShapeDtypeStruct((NPAD, 64), _f32),
          jax.ShapeDtypeStruct((NPAD, 16), _f32),
          jax.ShapeDtypeStruct((NPAD, 16), _f32),
          jax.ShapeDtypeStruct((NPAD, 16), _f32),
      ],
  )(denom1, msg1, h1p, wself1, b1_row, W2p, a2s_row, a2d_row, R8)

  denom2, msg2 = _make_sc_edge2()(src_p, dst_p, b2tab, h2tab)

  logits = pl.pallas_call(
      _k9_body,
      grid=(_GRID,),
      in_specs=[pl.BlockSpec((NC, _BLK, 16), lambda i: (0, i, 0)),
                pl.BlockSpec((NC, _BLK, 16), lambda i: (0, i, 0)),
                _tc_spec(16), _tc_spec(16), _row_spec(16)],
      out_specs=_tc_spec(16),
      out_shape=jax.ShapeDtypeStruct((NPAD, 16), _f32),
  )(denom2, msg2, h2tab, wself2, b2_row)

  return logits[:N, :C2], x1p[:N]


# speedup vs baseline: 1.0030x; 1.0030x over previous
import functools

import jax
import jax.numpy as jnp
from jax import lax
from jax.experimental import pallas as pl
from jax.experimental.pallas import tpu as pltpu
from jax.experimental.pallas import tpu_sc as plsc

N = 10000
NPAD = 10240
E = 320000
NC, NS = 2, 16
NW = NC * NS
CHUNK = 128
CPW = 80
EPAD = NW * CPW * CHUNK
RPT = NPAD // NS
H1, C1 = 8, 8
C2 = 10

_f32 = jnp.float32
_i32 = jnp.int32


@functools.lru_cache(maxsize=None)
def _sc_mesh():
  return plsc.VectorSubcoreMesh(
      core_axis_name="c", subcore_axis_name="s", num_cores=NC, num_subcores=NS)


def _sc_compiler_params():
  return pltpu.CompilerParams(
      use_tc_tiling_on_sc=False, needs_layout_passes=False)


def _iota16():
  return lax.iota(_i32, 16)


def _splat16(v):
  return jnp.broadcast_to(v, (16,)).astype(_i32)


def _zero_rows(buf, nrows, width_groups):
  z = jnp.zeros((16,), _f32)

  def body(i, _):
    if width_groups is None:
      buf[i] = z
    else:
      for g in range(width_groups):
        buf[i, g] = z
    return 0

  lax.fori_loop(0, nrows, body, 0)


def _fake_wait(src_hbm_like, dst_buf, sem):
  pltpu.make_async_copy(src_hbm_like, dst_buf, sem).wait()


def _run_pipeline(start_gathers, wait_gathers, compute, start_scatter,
                  wait_scatter):
  for b in range(2):
    start_gathers(b, b)

  def loop_body(j, _):
    for b in range(2):
      q = 2 * j + b
      wait_gathers(b)

      @pl.when(j > 0)
      def _():
        wait_scatter(b)

      compute(b)
      start_scatter(q, b)
      start_gathers(q + 2, b)
    return 0

  lax.fori_loop(0, CPW // 2 - 1, loop_body, 0)
  for b in range(2):
    q = CPW - 2 + b
    wait_gathers(b)
    wait_scatter(b)
    compute(b)
    start_scatter(q, b)
  for b in range(2):
    wait_scatter(b)


@functools.lru_cache(maxsize=None)
def _make_sc_edge1():
  @functools.partial(
      pl.kernel,
      out_type=(jax.ShapeDtypeStruct((NC, NPAD, 16), _f32),
                jax.ShapeDtypeStruct((NC, NPAD, 4, 16), _f32)),
      mesh=_sc_mesh(),
      compiler_params=_sc_compiler_params(),
      scratch_types=[
          pltpu.VMEM((CPW, CHUNK), _i32),
          pltpu.VMEM((CPW, CHUNK), _i32),
          pltpu.VMEM((CHUNK, 16), _f32),
          pltpu.VMEM((CHUNK, 16), _f32),
          pltpu.VMEM((CHUNK, 16), _f32),
          pltpu.VMEM((CHUNK, 16), _f32),
          pltpu.VMEM((CHUNK, 2, 16), _i32),
          pltpu.VMEM((CHUNK, 2, 16), _i32),
          pltpu.VMEM((CHUNK, 16), _f32),
          pltpu.VMEM((CHUNK, 16), _f32),
          pltpu.VMEM((CHUNK, 4, 16), _f32),
          pltpu.VMEM((CHUNK, 4, 16), _f32),
          pltpu.VMEM_SHARED((NPAD, 16), _f32),
          pltpu.VMEM_SHARED((NPAD, 4, 16), _f32),
          pltpu.SemaphoreType.DMA,
          pltpu.SemaphoreType.DMA,
          pltpu.SemaphoreType.DMA,
          pltpu.SemaphoreType.DMA,
          pltpu.SemaphoreType.DMA,
          pltpu.SemaphoreType.DMA,
      ],
  )
  def sc_edge1(src_hbm, dst_hbm, atab_hbm, btab_hbm, h1_hbm,
               dout_hbm, mout_hbm,
               sidx_all, didx_all, arows0, arows1, brows0, brows1,
               hrows0, hrows1, wrows0, wrows1, obuf0, obuf1,
               acc_d, acc_m, gsem0, gsem1, dsem0, dsem1, msem0, msem1):
    c = lax.axis_index("c")
    s = lax.axis_index("s")
    wid = s * NC + c
    slots = ((arows0, brows0, hrows0, wrows0, obuf0, gsem0, dsem0, msem0),
             (arows1, brows1, hrows1, wrows1, obuf1, gsem1, dsem1, msem1))

    _zero_rows(wrows0, CHUNK, None)
    _zero_rows(obuf0, CHUNK, 4)
    for b in range(RPT // CHUNK):
      pltpu.sync_copy(wrows0, acc_d.at[pl.ds(s * RPT + b * CHUNK, CHUNK)])
      pltpu.sync_copy(obuf0, acc_m.at[pl.ds(s * RPT + b * CHUNK, CHUNK)])

    pltpu.sync_copy(src_hbm.at[pl.ds(wid * CPW, CPW)], sidx_all)
    pltpu.sync_copy(dst_hbm.at[pl.ds(wid * CPW, CPW)], didx_all)
    plsc.subcore_barrier()

    io16 = _iota16()
    scale_base = io16 >> 3

    def start_gathers(q, b):
      ar, br, hr, gs = slots[b][0], slots[b][1], slots[b][2], slots[b][5]
      pltpu.async_copy(atab_hbm.at[sidx_all.at[q]], ar, gs)
      pltpu.async_copy(btab_hbm.at[didx_all.at[q]], br, gs)
      pltpu.async_copy(h1_hbm.at[sidx_all.at[q]], hr, gs)

    def wait_gathers(b):
      ar, br, hr, gs = slots[b][0], slots[b][1], slots[b][2], slots[b][5]
      _fake_wait(atab_hbm.at[pl.ds(0, CHUNK)], ar, gs)
      _fake_wait(btab_hbm.at[pl.ds(0, CHUNK)], br, gs)
      _fake_wait(h1_hbm.at[pl.ds(0, CHUNK)], hr, gs)

    def compute(b):
      ar, br, hr, wr, ob = (slots[b][0], slots[b][1], slots[b][2],
                            slots[b][3], slots[b][4])

      @plsc.parallel_loop(0, CHUNK, unroll=4)
      def _(k):
        kk = _splat16(k)
        t = ar[k] + br[k]
        wr[k] = jnp.exp(jnp.maximum(t, 0.2 * t))
        for g2 in range(2):
          pb = plsc.bitcast(hr[k, g2], jnp.bfloat16)
          lo, hi = plsc.unpack(pb, format=plsc.PackFormat.INTERLEAVED)
          s_lo = plsc.load_gather(wr, [kk, 4 * g2 + scale_base])
          s_hi = plsc.load_gather(wr, [kk, 4 * g2 + 2 + scale_base])
          ob[k, 2 * g2] = lo * s_lo
          ob[k, 2 * g2 + 1] = hi * s_hi

    def start_scatter(q, b):
      wr, ob, ds, ms = slots[b][3], slots[b][4], slots[b][6], slots[b][7]
      pltpu.async_copy(wr, acc_d.at[didx_all.at[q]], ds, add=True)
      pltpu.async_copy(ob, acc_m.at[didx_all.at[q]], ms, add=True)

    def wait_scatter(b):
      wr, ob, ds, ms = slots[b][3], slots[b][4], slots[b][6], slots[b][7]
      _fake_wait(atab_hbm.at[pl.ds(0, CHUNK)], wr, ds)
      _fake_wait(mout_hbm.at[0, pl.ds(0, CHUNK)], ob, ms)

    _run_pipeline(start_gathers, wait_gathers, compute, start_scatter,
                  wait_scatter)
    plsc.subcore_barrier()

    for b in range(RPT // CHUNK):
      r0 = s * RPT + b * CHUNK
      pltpu.sync_copy(acc_d.at[pl.ds(r0, CHUNK)], wrows0)
      pltpu.sync_copy(wrows0, dout_hbm.at[c, pl.ds(r0, CHUNK)])
      pltpu.sync_copy(acc_m.at[pl.ds(r0, CHUNK)], obuf0)
      pltpu.sync_copy(obuf0, mout_hbm.at[c, pl.ds(r0, CHUNK)])

  return sc_edge1


@functools.lru_cache(maxsize=None)
def _make_sc_edge2():
  @functools.partial(
      pl.kernel,
      out_type=(jax.ShapeDtypeStruct((NC, NPAD, 16), _f32),
                jax.ShapeDtypeStruct((NC, NPAD, 16), _f32)),
      mesh=_sc_mesh(),
      compiler_params=_sc_compiler_params(),
      scratch_types=[
          pltpu.VMEM((CPW, CHUNK), _i32),
          pltpu.VMEM((CPW, CHUNK), _i32),
          pltpu.VMEM((CHUNK, 16), _f32),
          pltpu.VMEM((CHUNK, 16), _f32),
          pltpu.VMEM((CHUNK, 16), _f32),
          pltpu.VMEM((CHUNK, 16), _f32),
          pltpu.VMEM((CHUNK, 16), _f32),
          pltpu.VMEM((CHUNK, 16), _f32),
          pltpu.VMEM((CHUNK, 16), _f32),
          pltpu.VMEM((CHUNK, 16), _f32),
          pltpu.VMEM_SHARED((NPAD, 16), _f32),
          pltpu.VMEM_SHARED((NPAD, 16), _f32),
          pltpu.SemaphoreType.DMA,
          pltpu.SemaphoreType.DMA,
          pltpu.SemaphoreType.DMA,
          pltpu.SemaphoreType.DMA,
          pltpu.SemaphoreType.DMA,
          pltpu.SemaphoreType.DMA,
      ],
  )
  def sc_edge2(src_hbm, dst_hbm, btab_hbm, h2tab_hbm,
               dout_hbm, mout_hbm,
               sidx_all, didx_all, brows0, brows1,
               hrows0, hrows1, wrows0, wrows1, obuf0, obuf1,
               acc_d, acc_m, gsem0, gsem1, dsem0, dsem1, msem0, msem1):
    c = lax.axis_index("c")
    s = lax.axis_index("s")
    wid = s * NC + c
    i15 = _splat16(15)
    slots = ((brows0, hrows0, wrows0, obuf0, gsem0, dsem0, msem0),
             (brows1, hrows1, wrows1, obuf1, gsem1, dsem1, msem1))

    _zero_rows(wrows0, CHUNK, None)
    for b in range(RPT // CHUNK):
      pltpu.sync_copy(wrows0, acc_d.at[pl.ds(s * RPT + b * CHUNK, CHUNK)])
      pltpu.sync_copy(wrows0, acc_m.at[pl.ds(s * RPT + b * CHUNK, CHUNK)])

    pltpu.sync_copy(src_hbm.at[pl.ds(wid * CPW, CPW)], sidx_all)
    pltpu.sync_copy(dst_hbm.at[pl.ds(wid * CPW, CPW)], didx_all)
    plsc.subcore_barrier()

    def start_gathers(q, b):
      br, hr, gs = slots[b][0], slots[b][1], slots[b][4]
      pltpu.async_copy(btab_hbm.at[didx_all.at[q]], br, gs)
      pltpu.async_copy(h2tab_hbm.at[sidx_all.at[q]], hr, gs)

    def wait_gathers(b):
      br, hr, gs = slots[b][0], slots[b][1], slots[b][4]
      _fake_wait(btab_hbm.at[pl.ds(0, CHUNK)], br, gs)
      _fake_wait(h2tab_hbm.at[pl.ds(0, CHUNK)], hr, gs)

    def compute(b):
      br, hr, wr, ob = (slots[b][0], slots[b][1], slots[b][2], slots[b][3])

      @plsc.parallel_loop(0, CHUNK, unroll=8)
      def _(k):
        kk = _splat16(k)
        asp = plsc.load_gather(hr, [kk, i15])
        t = asp + br[k]
        w = jnp.exp(jnp.maximum(t, 0.2 * t))
        wr[k] = w
        ob[k] = hr[k] * w

    def start_scatter(q, b):
      wr, ob, ds, ms = slots[b][2], slots[b][3], slots[b][5], slots[b][6]
      pltpu.async_copy(wr, acc_d.at[didx_all.at[q]], ds, add=True)
      pltpu.async_copy(ob, acc_m.at[didx_all.at[q]], ms, add=True)

    def wait_scatter(b):
      wr, ob, ds, ms = slots[b][2], slots[b][3], slots[b][5], slots[b][6]
      _fake_wait(btab_hbm.at[pl.ds(0, CHUNK)], wr, ds)
      _fake_wait(btab_hbm.at[pl.ds(0, CHUNK)], ob, ms)

    _run_pipeline(start_gathers, wait_gathers, compute, start_scatter,
                  wait_scatter)
    plsc.subcore_barrier()

    for b in range(RPT // CHUNK):
      r0 = s * RPT + b * CHUNK
      pltpu.sync_copy(acc_d.at[pl.ds(r0, CHUNK)], wrows0)
      pltpu.sync_copy(wrows0, dout_hbm.at[c, pl.ds(r0, CHUNK)])
      pltpu.sync_copy(acc_m.at[pl.ds(r0, CHUNK)], wrows0)
      pltpu.sync_copy(wrows0, mout_hbm.at[c, pl.ds(r0, CHUNK)])

  return sc_edge2


_BLK = 1024
_GRID = NPAD // _BLK


def _tc_spec(width):
  return pl.BlockSpec((_BLK, width), lambda i: (i, 0))


def _row_spec(width):
  return pl.BlockSpec((_BLK, width), lambda i: (0, 0))


def _full_spec(a):
  return pl.BlockSpec(a.shape, lambda i: tuple(0 for _ in a.shape))


def _k1_body(x_ref, w1_ref, as_ref, ad_ref,
             h1_ref, ph1_ref, s1tab_ref, adtab_ref, wself_ref):
  h = jnp.dot(x_ref[...], w1_ref[...], preferred_element_type=_f32)
  h1_ref[...] = h
  hb = h.astype(jnp.bfloat16)
  words = []
  for g2 in range(2):
    lo = lax.bitcast_convert_type(hb[:, 32 * g2:32 * g2 + 16],
                                  jnp.uint16).astype(jnp.uint32)
    hi = lax.bitcast_convert_type(hb[:, 32 * g2 + 16:32 * g2 + 32],
                                  jnp.uint16).astype(jnp.uint32)
    words.append(lo | (hi << 16))
  ph1_ref[...] = lax.bitcast_convert_type(
      jnp.concatenate(words, axis=1), jnp.int32)
  a_s = jnp.dot(h, as_ref[...], preferred_element_type=_f32)
  a_d = jnp.dot(h, ad_ref[...], preferred_element_type=_f32)
  s1tab_ref[...] = jnp.concatenate([a_s, a_s], axis=1)
  adtab_ref[...] = jnp.concatenate([a_d, a_d], axis=1)
  t = a_s + a_d
  wself_ref[...] = jnp.exp(jnp.maximum(t, 0.2 * t))


def _k5_body(dp_ref, mp_ref, h1_ref, wself_ref, b1_ref, w2_ref, a2s_ref,
             a2d_ref, r8_ref,
             x1_ref, h2tab_ref, b2tab_ref, wself2_ref):
  recip1 = 1.0 / (dp_ref[0][:, :8] + dp_ref[1][:, :8] + wself_ref[...] + 1e-16)
  r = jnp.dot(recip1, r8_ref[...], preferred_element_type=_f32)
  m = jnp.dot(wself_ref[...] * recip1, r8_ref[...],
              preferred_element_type=_f32)
  out1 = (mp_ref[0] + mp_ref[1]) * r + h1_ref[...] * m + b1_ref[...]
  x1 = jnp.where(out1 > 0, out1, jnp.exp(jnp.minimum(out1, 0.0)) - 1.0)
  x1_ref[...] = x1
  h2 = jnp.dot(x1, w2_ref[...], preferred_element_type=_f32)
  as2 = jnp.sum(h2 * a2s_ref[...], axis=1, keepdims=True)
  ad2 = jnp.sum(h2 * a2d_ref[...], axis=1, keepdims=True)
  lane = lax.broadcasted_iota(_i32, h2.shape, 1)
  h2tab_ref[...] = jnp.where(lane == 15, as2, h2)
  b2tab_ref[...] = jnp.broadcast_to(ad2, h2.shape)
  t = as2 + ad2
  wself2_ref[...] = jnp.broadcast_to(jnp.exp(jnp.maximum(t, 0.2 * t)), h2.shape)


def _k9_body(dp_ref, mp_ref, h2tab_ref, wself2_ref, b2_ref, out_ref):
  recip2 = 1.0 / (dp_ref[0] + dp_ref[1] + wself2_ref[...] + 1e-16)
  lane = lax.broadcasted_iota(_i32, recip2.shape, 1)
  h2 = jnp.where(lane == 15, 0.0, h2tab_ref[...])
  z = ((mp_ref[0] + mp_ref[1]) * recip2
       + h2 * (wself2_ref[...] * recip2) + b2_ref[...])
  valid = lane < C2
  zm = jnp.where(valid, z, -jnp.inf)
  mx = jnp.max(zm, axis=1, keepdims=True)
  ez = jnp.where(valid, jnp.exp(z - mx), 0.0)
  ssum = jnp.sum(ez, axis=1, keepdims=True)
  out_ref[...] = z - mx - jnp.log(ssum)


def kernel(x, edge_index, W1, a_src1, a_dst1, b1, W2, a_src2, a_dst2, b2):
  src = edge_index[0].astype(_i32)
  dst = edge_index[1].astype(_i32)
  pad_e = EPAD - E
  pad_idx = jnp.full((pad_e,), NPAD - 1, _i32)
  src_p = jnp.concatenate([src, pad_idx]).reshape(NW * CPW, CHUNK)
  dst_p = jnp.concatenate([dst, pad_idx]).reshape(NW * CPW, CHUNK)
  x_p = jnp.pad(x, ((0, NPAD - N), (0, 0)))

  eye8 = jnp.eye(H1, dtype=_f32)
  As1 = (a_src1[:, :, None] * eye8[:, None, :]).reshape(H1 * C1, H1)
  Ad1 = (a_dst1[:, :, None] * eye8[:, None, :]).reshape(H1 * C1, H1)
  R8 = (eye8[:, :, None] * jnp.ones((1, 1, C1), _f32)).reshape(H1, H1 * C1)
  b1_row = jnp.broadcast_to(b1[None, :], (_BLK, H1 * C1))
  W2p = jnp.pad(W2, ((0, 0), (0, 16 - C2)))
  a2s_row = jnp.broadcast_to(jnp.pad(a_src2[0], (0, 16 - C2))[None, :],
                             (_BLK, 16))
  a2d_row = jnp.broadcast_to(jnp.pad(a_dst2[0], (0, 16 - C2))[None, :],
                             (_BLK, 16))
  b2_row = jnp.broadcast_to(jnp.pad(b2, (0, 16 - C2))[None, :], (_BLK, 16))

  h1p, ph1, s1tab, adtab, wself1 = pl.pallas_call(
      _k1_body,
      grid=(_GRID,),
      in_specs=[_tc_spec(128), _full_spec(W1), _full_spec(As1), _full_spec(Ad1)],
      out_specs=[_tc_spec(64), _tc_spec(32), _tc_spec(16), _tc_spec(16),
                 _tc_spec(8)],
      out_shape=[
          jax.ShapeDtypeStruct((NPAD, 64), _f32),
          jax.ShapeDtypeStruct((NPAD, 32), _i32),
          jax.ShapeDtypeStruct((NPAD, 16), _f32),
          jax.ShapeDtypeStruct((NPAD, 16), _f32),
          jax.ShapeDtypeStruct((NPAD, 8), _f32),
      ],
  )(x_p, W1, As1, Ad1)

  denom1, msg1 = _make_sc_edge1()(src_p, dst_p, s1tab, adtab,
                                  ph1.reshape(NPAD, 2, 16))
  msg1 = msg1.reshape(NC, NPAD, 64)

  x1p, h2tab, b2tab, wself2 = pl.pallas_call(
      _k5_body,
      grid=(_GRID,),
      in_specs=[pl.BlockSpec((NC, _BLK, 16), lambda i: (0, i, 0)),
                pl.BlockSpec((NC, _BLK, 64), lambda i: (0, i, 0)),
                _tc_spec(64), _tc_spec(8), _row_spec(64),
                _full_spec(W2p), _row_spec(16), _row_spec(16), _full_spec(R8)],
      out_specs=[_tc_spec(64), _tc_spec(16), _tc_spec(16), _tc_spec(16)],
      out_shape=[
          jax.ShapeDtypeStruct((NPAD, 64), _f32),
          jax.ShapeDtypeStruct((NPAD, 16), _f32),
          jax.ShapeDtypeStruct((NPAD, 16), _f32),
          jax.ShapeDtypeStruct((NPAD, 16), _f32),
      ],
  )(denom1, msg1, h1p, wself1, b1_row, W2p, a2s_row, a2d_row, R8)

  denom2, msg2 = _make_sc_edge2()(src_p, dst_p, b2tab, h2tab)

  logits = pl.pallas_call(
      _k9_body,
      grid=(_GRID,),
      in_specs=[pl.BlockSpec((NC, _BLK, 16), lambda i: (0, i, 0)),
                pl.BlockSpec((NC, _BLK, 16), lambda i: (0, i, 0)),
                _tc_spec(16), _tc_spec(16), _row_spec(16)],
      out_specs=_tc_spec(16),
      out_shape=jax.ShapeDtypeStruct((NPAD, 16), _f32),
  )(denom2, msg2, h2tab, wself2, b2_row)

  return logits[:N, :C2], x1p[:N]
